# Initial kernel scaffold; baseline (speedup 1.0000x reference)
import jax
import jax.numpy as jnp
from jax.experimental import pallas as pl


def kernel(x, adj, W1, b1, W2, b2, W3, b3, Wl1, bl1, Wl2, bl2, Wl3, bl3):
    raise NotImplementedError("write your pallas kernel here")


# speedup vs baseline: 1.1964x; 1.1964x over previous
import jax
import jax.numpy as jnp
from jax.experimental import pallas as pl

N = 10000
DIM_TOUCHED = 32
NN = 1000
BR = 400
GRID = N // BR
GROUP = 640
NPAD = 10240
SEL_ROWS = NPAD // 128


def _dot(a, b):
    return jnp.dot(a, b, preferred_element_type=jnp.float32)


def _gcn_acc(adj_ref, g_ref):
    acc = None
    for s in range(0, N, GROUP):
        e1 = min(s + 256, N)
        e2 = min(s + GROUP, N)
        a = _dot(adj_ref[:, s:e1], g_ref[s:e1])
        b = _dot(adj_ref[:, e1:e2], g_ref[e1:e2]) if e2 > e1 else None
        t = (b + a) if b is not None else a
        acc = t if acc is None else acc + t
    return acc


def _g1_kernel(x32_ref, w1_ref, g1_ref):
    g1_ref[...] = _dot(x32_ref[...], w1_ref[...])


def _layer_kernel(adj_ref, g_ref, b_ref, wn_ref, out_ref):
    h = jnp.maximum(_gcn_acc(adj_ref, g_ref) + b_ref[...], 0.0)
    out_ref[...] = _dot(h, wn_ref[...])


def _final_kernel(adj_ref, g_ref, b3_ref, xmid_ref, wl1_ref, bl1_ref,
                  wl2_ref, bl2_ref, wl3_ref, bl3_ref, m_ref):
    h3 = jnp.maximum(_gcn_acc(adj_ref, g_ref) + b3_ref[...], 0.0)
    feat = jnp.concatenate([h3, xmid_ref[...]], axis=1)
    m1 = jnp.maximum(_dot(feat, wl1_ref[...]) + bl1_ref[...], 0.0)
    m2 = jnp.maximum(_dot(m1, wl2_ref[...]) + bl2_ref[...], 0.0)
    m_ref[...] = _dot(m2, wl3_ref[...]) + bl3_ref[...]


def _select_kernel(m_ref, flag_ref, out_ref):
    m = m_ref[...]
    flag = flag_ref[...]
    mn = jnp.min(m)
    mm = jnp.where(flag == 0.0, mn, m)
    u = jax.lax.bitcast_convert_type(mm, jnp.uint32)
    mask = jnp.where(u >= jnp.uint32(0x80000000),
                     jnp.uint32(0xFFFFFFFF), jnp.uint32(0x80000000))
    ukey = u ^ mask

    k = NN + 1

    def body(i, prefix):
        b = (31 - i).astype(jnp.uint32)
        cand = prefix | jax.lax.shift_left(jnp.uint32(1), b)
        c = jnp.sum((ukey >= cand).astype(jnp.int32))
        return jnp.where(c >= k, cand, prefix)

    thr_u = jax.lax.fori_loop(0, 32, body, jnp.uint32(0))
    thr_bits = jnp.where(thr_u >= jnp.uint32(0x80000000),
                         thr_u ^ jnp.uint32(0x80000000), ~thr_u)
    thr = jax.lax.bitcast_convert_type(thr_bits, jnp.float32)
    recip = jnp.reciprocal(mm)
    out_ref[...] = mm * jnp.where(mm > thr, recip, 0.0)


def kernel(x, adj, W1, b1, W2, b2, W3, b3, Wl1, bl1, Wl2, bl2, Wl3, bl3):
    x32 = x[:, :DIM_TOUCHED]
    xmid = x[:, DIM_TOUCHED:-1]
    xflag = x[:, -1]
    b1r = b1.reshape(1, -1)
    b2r = b2.reshape(1, -1)
    b3r = b3.reshape(1, -1)
    bl1r = bl1.reshape(1, -1)
    bl2r = bl2.reshape(1, -1)
    bl3r = bl3.reshape(1, -1)

    g1 = pl.pallas_call(
        _g1_kernel,
        out_shape=jax.ShapeDtypeStruct((N, 16), jnp.float32),
    )(x32, W1)

    full = lambda shape: pl.BlockSpec(shape, lambda i: (0, 0))
    rows = lambda w: pl.BlockSpec((BR, w), lambda i: (i, 0))

    def layer(g, b, wn):
        return pl.pallas_call(
            _layer_kernel,
            grid=(GRID,),
            in_specs=[rows(N), full((N, 16)), full((1, 16)), full((16, 16))],
            out_specs=rows(16),
            out_shape=jax.ShapeDtypeStruct((N, 16), jnp.float32),
        )(adj, g, b, wn)

    g2 = layer(g1, b1r, W2)
    g3 = layer(g2, b2r, W3)

    m = pl.pallas_call(
        _final_kernel,
        grid=(GRID,),
        in_specs=[rows(N), full((N, 16)), full((1, 16)), rows(31),
                  full((47, 64)), full((1, 64)),
                  full((64, 32)), full((1, 32)), full((32, 1)),
                  full((1, 1))],
        out_specs=rows(1),
        out_shape=jax.ShapeDtypeStruct((N, 1), jnp.float32),
    )(adj, g3, b3r, xmid, Wl1, bl1r, Wl2, bl2r, Wl3, bl3r)

    m_pad = jnp.pad(m[:, 0], (0, NPAD - N),
                    constant_values=jnp.inf).reshape(SEL_ROWS, 128)
    flag_pad = jnp.pad(xflag, (0, NPAD - N),
                       constant_values=0.0).reshape(SEL_ROWS, 128)

    out = pl.pallas_call(
        _select_kernel,
        out_shape=jax.ShapeDtypeStruct((SEL_ROWS, 128), jnp.float32),
    )(m_pad, flag_pad)

    return out.reshape(-1)[:N][:, None]
